# Initial kernel scaffold; baseline (speedup 1.0000x reference)
import jax
import jax.numpy as jnp
from jax.experimental import pallas as pl


def kernel(input):
    raise NotImplementedError("write your pallas kernel here")


# speedup vs baseline: 1.4542x; 1.4542x over previous
import jax
import jax.numpy as jnp
from jax.experimental import pallas as pl
from jax.experimental.pallas import tpu as pltpu

_EPS = 1e-05
_BLOCK_ROWS = 512


def _l1norm_body(x_ref, o_ref):
    blk = x_ref[...]
    row_sum = jnp.sum(blk, axis=1, keepdims=True)
    inv = 1.0 / jnp.maximum(row_sum, _EPS)
    o_ref[...] = blk * inv


def kernel(input):
    bs, r, d = input.shape
    x = input.reshape(bs * r, d)
    n_rows = bs * r
    grid = (n_rows // _BLOCK_ROWS,)
    out = pl.pallas_call(
        _l1norm_body,
        grid=grid,
        in_specs=[pl.BlockSpec((_BLOCK_ROWS, d), lambda i: (i, 0))],
        out_specs=pl.BlockSpec((_BLOCK_ROWS, d), lambda i: (i, 0)),
        out_shape=jax.ShapeDtypeStruct((n_rows, d), x.dtype),
        compiler_params=pltpu.CompilerParams(
            dimension_semantics=("parallel",),
            vmem_limit_bytes=56 * 1024 * 1024,
        ),
        name="l1_row_norm",
    )(x)
    return out.reshape(bs, r, d)
